# baseline (speedup 1.0000x reference)
import functools
import jax
import jax.numpy as jnp
from jax import lax
from jax.experimental import pallas as pl
from jax.experimental.pallas import tpu as pltpu
from jax.experimental.pallas import tpu_sc as plsc

N = 10000
NPAD = 10240
D = 128
E = 320000
EPAD = 327680
NW = 32
CHUNK = 64
NCHUNKS = EPAD // CHUNK
PHASES = 5
NBUF = 4
CW0 = 80
CW1 = 240
PHMAX = CW1 // PHASES
RS = 624
RTAIL = 16
ACCROWS = N + 16
DEGW = D
BLK = 1024
GRID = NPAD // BLK
NG = 64

def _acc_init(zrows_hbm, acc_sh, s):
    pltpu.sync_copy(zrows_hbm.at[pl.ds(0, RS)], acc_sh.at[pl.ds(s * RS, RS)])

    @pl.when(s == 0)
    def _():
        pltpu.sync_copy(zrows_hbm.at[pl.ds(0, RTAIL)],
                        acc_sh.at[pl.ds(16 * RS, RTAIL)])


def _acc_writeback(acc_sh, out_hbm, c, s):
    pltpu.sync_copy(acc_sh.at[pl.ds(s * RS, RS)],
                    out_hbm.at[c, pl.ds(s * RS, RS)])

    @pl.when(s == 0)
    def _():
        pltpu.sync_copy(acc_sh.at[pl.ds(16 * RS, RTAIL)],
                        out_hbm.at[c, pl.ds(16 * RS, RTAIL)])


def _scatter_sc_body(hp_hbm, src_hbm, dst_hbm, zrows_hbm, out_hbm,
                     src_v, dst_v, rows0, rows1, rows2, rows3, acc_sh,
                     g0, g1, g2, g3, s0, s1, s2, s3):
    c = lax.axis_index("c")
    s = lax.axis_index("s")
    cw = jnp.where(c == 0, CW0, CW1)
    ph = jnp.where(c == 0, CW0 // PHASES, CW1 // PHASES)
    trip = jnp.where(c == 0, CW0 // PHASES // NBUF, CW1 // PHASES // NBUF)
    wbase = jnp.where(c == 0, 0, 16 * CW0) + s * cw
    _acc_init(zrows_hbm, acc_sh, s)
    plsc.subcore_barrier()

    rows = (rows0, rows1, rows2, rows3)
    gsem = (g0, g1, g2, g3)
    ssem = (s0, s1, s2, s3)

    def gather(j, b):
        pltpu.async_copy(hp_hbm.at[src_v.at[j]], rows[b], gsem[b])

    def gather_wait(j, b):
        pltpu.make_async_copy(hp_hbm.at[src_v.at[j]], rows[b],
                              gsem[b]).wait()

    def scat(j, b):
        pltpu.async_copy(rows[b], acc_sh.at[dst_v.at[j]], ssem[b], add=True)

    def scat_wait(j, b):
        del j
        pltpu.make_async_copy(rows[b], acc_sh.at[pl.ds(0, CHUNK)],
                              ssem[b]).wait()

    for p in range(PHASES):
        base = wbase + p * ph
        pltpu.sync_copy(src_hbm.at[pl.ds(base, PHMAX)], src_v)
        pltpu.sync_copy(dst_hbm.at[pl.ds(base, PHMAX)], dst_v)
        gather(0, 0)
        gather(1, 1)

        def body(i, carry):
            for b in range(NBUF):
                j = NBUF * i + b
                gather_wait(j, b)
                scat(j, b)
                f = j + 2
                bf = (b + 2) % NBUF

                @pl.when(f < ph)
                def _():
                    @pl.when(f >= NBUF)
                    def _():
                        scat_wait(f - NBUF, bf)

                    gather(f, bf)
            return carry

        lax.fori_loop(0, trip, body, 0)
        for b in range(NBUF):
            scat_wait(0, b)

    plsc.subcore_barrier()
    _acc_writeback(acc_sh, out_hbm, c, s)


@functools.lru_cache(maxsize=None)
def _sc_kernels():
    mesh = plsc.VectorSubcoreMesh(core_axis_name="c", subcore_axis_name="s")
    scatter_sc = functools.partial(
        pl.kernel,
        out_type=jax.ShapeDtypeStruct((2, N, D), jnp.float32),
        mesh=mesh,
        scratch_types=(
            [pltpu.VMEM((PHMAX, CHUNK), jnp.int32)] * 2
            + [pltpu.VMEM((CHUNK, D), jnp.float32)] * NBUF
            + [pltpu.VMEM_SHARED((ACCROWS, D), jnp.float32)]
            + [pltpu.SemaphoreType.DMA] * (2 * NBUF)
        ),
    )(_scatter_sc_body)
    return scatter_sc


def _first_body(x_ref, degp_ref, w_ref, hp_ref, dis_ref):
    i = pl.program_id(0)
    row = i * BLK + lax.broadcasted_iota(jnp.int32, (BLK, 1), 0)
    valid = row < N
    deg = degp_ref[0, :, 0:1] + degp_ref[1, :, 0:1] + 1.0
    dis = jnp.where(valid, lax.rsqrt(deg), 0.0)
    dis_ref[...] = dis
    h = jnp.dot(x_ref[...], w_ref[...], preferred_element_type=jnp.float32)
    hp_ref[...] = jnp.where(valid, dis * h, 0.0)


_first_tc = pl.pallas_call(
    _first_body,
    grid=(GRID,),
    in_specs=[
        pl.BlockSpec((BLK, D), lambda i: (i, 0)),
        pl.BlockSpec((2, BLK, D), lambda i: (0, i, 0)),
        pl.BlockSpec((D, D), lambda i: (0, 0)),
    ],
    out_specs=[
        pl.BlockSpec((BLK, D), lambda i: (i, 0)),
        pl.BlockSpec((BLK, 1), lambda i: (i, 0)),
    ],
    out_shape=[
        jax.ShapeDtypeStruct((NPAD, D), jnp.float32),
        jax.ShapeDtypeStruct((NPAD, 1), jnp.float32),
    ],
)


def _fuse_body(s_ref, hp_ref, dis_ref, b_ref, w_ref, out_ref):
    i = pl.program_id(0)
    row = i * BLK + lax.broadcasted_iota(jnp.int32, (BLK, 1), 0)
    valid = row < N
    dis = dis_ref[...]
    pre = dis * (s_ref[0] + s_ref[1] + hp_ref[...]) + b_ref[...]
    h = jnp.where(valid, jnp.maximum(pre, 0.0), 0.0)
    hn = jnp.dot(h, w_ref[...], preferred_element_type=jnp.float32)
    out_ref[...] = jnp.where(valid, dis * hn, 0.0)


_fuse_tc = pl.pallas_call(
    _fuse_body,
    grid=(GRID,),
    in_specs=[
        pl.BlockSpec((2, BLK, D), lambda i: (0, i, 0)),
        pl.BlockSpec((BLK, D), lambda i: (i, 0)),
        pl.BlockSpec((BLK, 1), lambda i: (i, 0)),
        pl.BlockSpec((1, D), lambda i: (0, 0)),
        pl.BlockSpec((D, D), lambda i: (0, 0)),
    ],
    out_specs=pl.BlockSpec((BLK, D), lambda i: (i, 0)),
    out_shape=jax.ShapeDtypeStruct((NPAD, D), jnp.float32),
)


def _last_body(s_ref, hp_ref, dis_ref, b_ref, out_ref):
    i = pl.program_id(0)
    row = i * BLK + lax.broadcasted_iota(jnp.int32, (BLK, 1), 0)
    valid = row < N
    dis = dis_ref[...]
    pre = dis * (s_ref[0] + s_ref[1] + hp_ref[...]) + b_ref[...]
    out_ref[...] = jnp.where(valid, jnp.maximum(pre, 0.0), 0.0)


_last_tc = pl.pallas_call(
    _last_body,
    grid=(GRID,),
    in_specs=[
        pl.BlockSpec((2, BLK, D), lambda i: (0, i, 0)),
        pl.BlockSpec((BLK, D), lambda i: (i, 0)),
        pl.BlockSpec((BLK, 1), lambda i: (i, 0)),
        pl.BlockSpec((1, D), lambda i: (0, 0)),
    ],
    out_specs=pl.BlockSpec((BLK, D), lambda i: (i, 0)),
    out_shape=jax.ShapeDtypeStruct((NPAD, D), jnp.float32),
)


def _pool_body(h_ref, bat_ref, w1_ref, b1_ref, w2_ref, b2_ref,
               w3_ref, b3_ref, out_ref, m_ref):
    m_ref[...] = jnp.full((NG, D), -jnp.inf, jnp.float32)

    def blk(bi, carry):
        rows = h_ref[pl.ds(bi * 128, 128), :]
        ids = bat_ref[pl.ds(bi * 128, 128), :]
        lo = jnp.minimum(jnp.min(ids), NG - 1)
        hi = jnp.minimum(jnp.max(ids), NG - 1)

        def g_body(g, c2):
            msk = ids == g
            vals = jnp.where(msk, rows, -jnp.inf)
            mg = jnp.max(vals, axis=0, keepdims=True)
            m_ref[pl.ds(g, 1), :] = jnp.maximum(m_ref[pl.ds(g, 1), :], mg)
            return c2

        return lax.fori_loop(lo, hi + 1, g_body, carry)

    lax.fori_loop(0, NPAD // 128, blk, 0)
    g1 = jnp.dot(m_ref[...], w1_ref[...], preferred_element_type=jnp.float32)
    g1 = g1 + b1_ref[...]
    g2 = jnp.dot(g1, w2_ref[...], preferred_element_type=jnp.float32)
    g2 = g2 + b2_ref[...]
    g3 = jnp.dot(g2, w3_ref[...], preferred_element_type=jnp.float32)
    out_ref[...] = g3 + b3_ref[...]


_pool_tc = pl.pallas_call(
    _pool_body,
    out_shape=jax.ShapeDtypeStruct((NG, 2), jnp.float32),
    scratch_shapes=[pltpu.VMEM((NG, D), jnp.float32)],
)


def kernel(x, edge_index, batch, W1, b1, W2, b2, W3, b3, W4, b4,
           lin1_W, lin1_b, lin2_W, lin2_b, out_W, out_b):
    src = edge_index[0].astype(jnp.int32)
    dst = edge_index[1].astype(jnp.int32)
    npad_e = EPAD - E
    srcp = jnp.concatenate([src, jnp.full((npad_e,), N, jnp.int32)])
    dstp = jnp.concatenate([dst, jnp.full((npad_e,), N, jnp.int32)])
    srcp = srcp.reshape(NCHUNKS, CHUNK)
    dstp = dstp.reshape(NCHUNKS, CHUNK)
    batchp = jnp.concatenate(
        [batch.astype(jnp.int32), jnp.full((NPAD - N,), NG, jnp.int32)]
    ).reshape(NPAD, 1)
    ones_table = jnp.concatenate(
        [jnp.ones((N, D), jnp.float32), jnp.zeros((NPAD - N, D), jnp.float32)])
    zrows = jnp.zeros((RS, D), jnp.float32)

    _scatter_sc = _sc_kernels()
    degp = _scatter_sc(ones_table, srcp, dstp, zrows)
    hp, dis = _first_tc(x, degp, W1)
    for b_l, W_next in ((b1, W2), (b2, W3), (b3, W4)):
        S = _scatter_sc(hp, srcp, dstp, zrows)
        hp = _fuse_tc(S, hp, dis, b_l.reshape(1, D), W_next)
    S = _scatter_sc(hp, srcp, dstp, zrows)
    h4 = _last_tc(S, hp, dis, b4.reshape(1, D))

    return _pool_tc(h4, batchp, lin1_W, lin1_b.reshape(1, -1),
                    lin2_W, lin2_b.reshape(1, -1), out_W, out_b.reshape(1, -1))


# speedup vs baseline: 1.3347x; 1.3347x over previous
import functools
import jax
import jax.numpy as jnp
from jax import lax
from jax.experimental import pallas as pl
from jax.experimental.pallas import tpu as pltpu
from jax.experimental.pallas import tpu_sc as plsc

N = 10000
NPAD = 10240
D = 128
E = 320000
EPAD = 327680
NW = 32
CHUNK = 64
NCHUNKS = EPAD // CHUNK
PHASES = 5
NBUF = 4
CW0 = 240
CW1 = 80
PHMAX = CW0 // PHASES
RS = 624
RTAIL = 16
ACCROWS = N + 16
DEGW = D
BLK = 1024
GRID = NPAD // BLK
NG = 64

def _acc_init(zrows_hbm, acc_sh, s):
    pltpu.sync_copy(zrows_hbm.at[pl.ds(0, RS)], acc_sh.at[pl.ds(s * RS, RS)])

    @pl.when(s == 0)
    def _():
        pltpu.sync_copy(zrows_hbm.at[pl.ds(0, RTAIL)],
                        acc_sh.at[pl.ds(16 * RS, RTAIL)])


def _acc_writeback(acc_sh, out_hbm, c, s):
    pltpu.sync_copy(acc_sh.at[pl.ds(s * RS, RS)],
                    out_hbm.at[c, pl.ds(s * RS, RS)])

    @pl.when(s == 0)
    def _():
        pltpu.sync_copy(acc_sh.at[pl.ds(16 * RS, RTAIL)],
                        out_hbm.at[c, pl.ds(16 * RS, RTAIL)])


def _scatter_sc_body(hp_hbm, src_hbm, dst_hbm, zrows_hbm, out_hbm,
                     src_v, dst_v, rows0, rows1, rows2, rows3, acc_sh,
                     g0, g1, g2, g3, s0, s1, s2, s3):
    c = lax.axis_index("c")
    s = lax.axis_index("s")
    cw = jnp.where(c == 0, CW0, CW1)
    ph = jnp.where(c == 0, CW0 // PHASES, CW1 // PHASES)
    trip = jnp.where(c == 0, CW0 // PHASES // NBUF, CW1 // PHASES // NBUF)
    wbase = jnp.where(c == 0, 0, 16 * CW0) + s * cw
    _acc_init(zrows_hbm, acc_sh, s)
    plsc.subcore_barrier()

    rows = (rows0, rows1, rows2, rows3)
    gsem = (g0, g1, g2, g3)
    ssem = (s0, s1, s2, s3)

    def gather(j, b):
        pltpu.async_copy(hp_hbm.at[src_v.at[j]], rows[b], gsem[b])

    def gather_wait(j, b):
        pltpu.make_async_copy(hp_hbm.at[src_v.at[j]], rows[b],
                              gsem[b]).wait()

    def scat(j, b):
        pltpu.async_copy(rows[b], acc_sh.at[dst_v.at[j]], ssem[b], add=True)

    def scat_wait(j, b):
        del j
        pltpu.make_async_copy(rows[b], acc_sh.at[pl.ds(0, CHUNK)],
                              ssem[b]).wait()

    for p in range(PHASES):
        base = wbase + p * ph
        pltpu.sync_copy(src_hbm.at[pl.ds(base, PHMAX)], src_v)
        pltpu.sync_copy(dst_hbm.at[pl.ds(base, PHMAX)], dst_v)
        gather(0, 0)
        gather(1, 1)

        def body(i, carry):
            for b in range(NBUF):
                j = NBUF * i + b
                gather_wait(j, b)
                scat(j, b)
                f = j + 2
                bf = (b + 2) % NBUF

                @pl.when(f < ph)
                def _():
                    @pl.when(f >= NBUF)
                    def _():
                        scat_wait(f - NBUF, bf)

                    gather(f, bf)
            return carry

        lax.fori_loop(0, trip, body, 0)
        for b in range(NBUF):
            scat_wait(0, b)

    plsc.subcore_barrier()
    _acc_writeback(acc_sh, out_hbm, c, s)


@functools.lru_cache(maxsize=None)
def _sc_kernels():
    mesh = plsc.VectorSubcoreMesh(core_axis_name="c", subcore_axis_name="s")
    scatter_sc = functools.partial(
        pl.kernel,
        out_type=jax.ShapeDtypeStruct((2, N, D), jnp.float32),
        mesh=mesh,
        scratch_types=(
            [pltpu.VMEM((PHMAX, CHUNK), jnp.int32)] * 2
            + [pltpu.VMEM((CHUNK, D), jnp.float32)] * NBUF
            + [pltpu.VMEM_SHARED((ACCROWS, D), jnp.float32)]
            + [pltpu.SemaphoreType.DMA] * (2 * NBUF)
        ),
    )(_scatter_sc_body)
    return scatter_sc


def _first_body(x_ref, degp_ref, w_ref, hp_ref, dis_ref):
    i = pl.program_id(0)
    row = i * BLK + lax.broadcasted_iota(jnp.int32, (BLK, 1), 0)
    valid = row < N
    deg = degp_ref[0, :, 0:1] + degp_ref[1, :, 0:1] + 1.0
    dis = jnp.where(valid, lax.rsqrt(deg), 0.0)
    dis_ref[...] = dis
    h = jnp.dot(x_ref[...], w_ref[...], preferred_element_type=jnp.float32)
    hp_ref[...] = jnp.where(valid, dis * h, 0.0)


_first_tc = pl.pallas_call(
    _first_body,
    grid=(GRID,),
    in_specs=[
        pl.BlockSpec((BLK, D), lambda i: (i, 0)),
        pl.BlockSpec((2, BLK, D), lambda i: (0, i, 0)),
        pl.BlockSpec((D, D), lambda i: (0, 0)),
    ],
    out_specs=[
        pl.BlockSpec((BLK, D), lambda i: (i, 0)),
        pl.BlockSpec((BLK, 1), lambda i: (i, 0)),
    ],
    out_shape=[
        jax.ShapeDtypeStruct((NPAD, D), jnp.float32),
        jax.ShapeDtypeStruct((NPAD, 1), jnp.float32),
    ],
)


def _fuse_body(s_ref, hp_ref, dis_ref, b_ref, w_ref, out_ref):
    i = pl.program_id(0)
    row = i * BLK + lax.broadcasted_iota(jnp.int32, (BLK, 1), 0)
    valid = row < N
    dis = dis_ref[...]
    pre = dis * (s_ref[0] + s_ref[1] + hp_ref[...]) + b_ref[...]
    h = jnp.where(valid, jnp.maximum(pre, 0.0), 0.0)
    hn = jnp.dot(h, w_ref[...], preferred_element_type=jnp.float32)
    out_ref[...] = jnp.where(valid, dis * hn, 0.0)


_fuse_tc = pl.pallas_call(
    _fuse_body,
    grid=(GRID,),
    in_specs=[
        pl.BlockSpec((2, BLK, D), lambda i: (0, i, 0)),
        pl.BlockSpec((BLK, D), lambda i: (i, 0)),
        pl.BlockSpec((BLK, 1), lambda i: (i, 0)),
        pl.BlockSpec((1, D), lambda i: (0, 0)),
        pl.BlockSpec((D, D), lambda i: (0, 0)),
    ],
    out_specs=pl.BlockSpec((BLK, D), lambda i: (i, 0)),
    out_shape=jax.ShapeDtypeStruct((NPAD, D), jnp.float32),
)


def _last_body(s_ref, hp_ref, dis_ref, b_ref, out_ref):
    i = pl.program_id(0)
    row = i * BLK + lax.broadcasted_iota(jnp.int32, (BLK, 1), 0)
    valid = row < N
    dis = dis_ref[...]
    pre = dis * (s_ref[0] + s_ref[1] + hp_ref[...]) + b_ref[...]
    out_ref[...] = jnp.where(valid, jnp.maximum(pre, 0.0), 0.0)


_last_tc = pl.pallas_call(
    _last_body,
    grid=(GRID,),
    in_specs=[
        pl.BlockSpec((2, BLK, D), lambda i: (0, i, 0)),
        pl.BlockSpec((BLK, D), lambda i: (i, 0)),
        pl.BlockSpec((BLK, 1), lambda i: (i, 0)),
        pl.BlockSpec((1, D), lambda i: (0, 0)),
    ],
    out_specs=pl.BlockSpec((BLK, D), lambda i: (i, 0)),
    out_shape=jax.ShapeDtypeStruct((NPAD, D), jnp.float32),
)


def _pool_body(h_ref, bat_ref, w1_ref, b1_ref, w2_ref, b2_ref,
               w3_ref, b3_ref, out_ref, m_ref):
    m_ref[...] = jnp.full((NG, D), -jnp.inf, jnp.float32)

    def blk(bi, carry):
        rows = h_ref[pl.ds(bi * 128, 128), :]
        ids = bat_ref[pl.ds(bi * 128, 128), :]
        lo = jnp.minimum(jnp.min(ids), NG - 1)
        hi = jnp.minimum(jnp.max(ids), NG - 1)

        def g_body(g, c2):
            msk = ids == g
            vals = jnp.where(msk, rows, -jnp.inf)
            mg = jnp.max(vals, axis=0, keepdims=True)
            m_ref[pl.ds(g, 1), :] = jnp.maximum(m_ref[pl.ds(g, 1), :], mg)
            return c2

        return lax.fori_loop(lo, hi + 1, g_body, carry)

    lax.fori_loop(0, NPAD // 128, blk, 0)
    g1 = jnp.dot(m_ref[...], w1_ref[...], preferred_element_type=jnp.float32)
    g1 = g1 + b1_ref[...]
    g2 = jnp.dot(g1, w2_ref[...], preferred_element_type=jnp.float32)
    g2 = g2 + b2_ref[...]
    g3 = jnp.dot(g2, w3_ref[...], preferred_element_type=jnp.float32)
    out_ref[...] = g3 + b3_ref[...]


_pool_tc = pl.pallas_call(
    _pool_body,
    out_shape=jax.ShapeDtypeStruct((NG, 2), jnp.float32),
    scratch_shapes=[pltpu.VMEM((NG, D), jnp.float32)],
)


def kernel(x, edge_index, batch, W1, b1, W2, b2, W3, b3, W4, b4,
           lin1_W, lin1_b, lin2_W, lin2_b, out_W, out_b):
    src = edge_index[0].astype(jnp.int32)
    dst = edge_index[1].astype(jnp.int32)
    npad_e = EPAD - E
    srcp = jnp.concatenate([src, jnp.full((npad_e,), N, jnp.int32)])
    dstp = jnp.concatenate([dst, jnp.full((npad_e,), N, jnp.int32)])
    tailz = jnp.zeros((PHMAX, CHUNK), jnp.int32)
    srcp = jnp.concatenate([srcp.reshape(NCHUNKS, CHUNK), tailz])
    dstp = jnp.concatenate([dstp.reshape(NCHUNKS, CHUNK), tailz])
    batchp = jnp.concatenate(
        [batch.astype(jnp.int32), jnp.full((NPAD - N,), NG, jnp.int32)]
    ).reshape(NPAD, 1)
    ones_table = jnp.concatenate(
        [jnp.ones((N, D), jnp.float32), jnp.zeros((NPAD - N, D), jnp.float32)])
    zrows = jnp.zeros((RS, D), jnp.float32)

    _scatter_sc = _sc_kernels()
    degp = _scatter_sc(ones_table, srcp, dstp, zrows)
    hp, dis = _first_tc(x, degp, W1)
    for b_l, W_next in ((b1, W2), (b2, W3), (b3, W4)):
        S = _scatter_sc(hp, srcp, dstp, zrows)
        hp = _fuse_tc(S, hp, dis, b_l.reshape(1, D), W_next)
    S = _scatter_sc(hp, srcp, dstp, zrows)
    h4 = _last_tc(S, hp, dis, b4.reshape(1, D))

    return _pool_tc(h4, batchp, lin1_W, lin1_b.reshape(1, -1),
                    lin2_W, lin2_b.reshape(1, -1), out_W, out_b.reshape(1, -1))
